# baseline (speedup 1.0000x reference)
import functools

import jax
import jax.numpy as jnp
from jax import lax
from jax.experimental import pallas as pl
from jax.experimental.pallas import tpu as pltpu
from jax.experimental.pallas import tpu_sc as plsc

B = 2
F = 128
N_IN = 10000
N_OUT = 5000
K = 32
D = B * F
NC = 2
NS = 16
DH = D // NC
DW = DH // 2
OUT_PAD = 5120
PER_S = OUT_PAD // NS
NB = 4
ROWS = NB * K
NBATCH = PER_S // NB
OGROUP = 4 * NB
N_IN_PAD = 10240
ROWS_PER_TILE = N_IN_PAD // NS
BLANES = 32


_mesh = plsc.VectorSubcoreMesh(core_axis_name="c", subcore_axis_name="s")


@functools.partial(
    pl.kernel,
    out_type=jax.ShapeDtypeStruct((NC, OUT_PAD, DW), jnp.int32),
    mesh=_mesh,
    scratch_types=[
        pltpu.VMEM((PER_S * K,), jnp.int32),
        pltpu.VMEM((ROWS, DW), jnp.int32),
        pltpu.VMEM((ROWS, DW), jnp.int32),
        pltpu.VMEM((OGROUP, DW), jnp.int32),
        pltpu.VMEM((OGROUP, DW), jnp.int32),
        pltpu.VMEM_SHARED((N_IN_PAD, DW), jnp.int32),
        pltpu.SemaphoreType.DMA,
        pltpu.SemaphoreType.DMA,
        pltpu.SemaphoreType.DMA,
        pltpu.SemaphoreType.DMA,
    ],
)
def _sc_gather_max(xt_hbm, idx_hbm, out_hbm, idx_v, gbuf0, gbuf1, obuf0,
                   obuf1, table, gsem0, gsem1, osem0, osem1):
    c = lax.axis_index("c")
    s = lax.axis_index("s")

    pltpu.sync_copy(
        xt_hbm.at[c, pl.ds(s * ROWS_PER_TILE, ROWS_PER_TILE), :],
        table.at[pl.ds(s * ROWS_PER_TILE, ROWS_PER_TILE)],
    )
    pltpu.sync_copy(idx_hbm.at[pl.ds(s * (PER_S * K), PER_S * K)], idx_v)
    plsc.subcore_barrier()

    gbufs = (gbuf0, gbuf1)
    gsems = (gsem0, gsem1)
    obufs = (obuf0, obuf1)
    osems = (osem0, osem1)

    def gather_src(j):
        return table.at[idx_v.at[pl.ds(j * ROWS, ROWS)]]

    def out_dst(g):
        return out_hbm.at[c, pl.ds(s * PER_S + g * OGROUP, OGROUP), :]

    def compute(buf, obuf, q):
        lo_mask = jnp.uint32(0xFFFF)
        hi_mask = jnp.uint32(0xFFFF0000)

        @plsc.parallel_loop(0, DW // 16)
        def col_body(d):
            col = d * 16
            for g in range(NB):
                r0 = g * K

                def ld(r):
                    return plsc.bitcast(
                        buf[r0 + r, pl.ds(col, 16)], jnp.uint32
                    )

                his = []
                los = []
                for i in range(K // 2):
                    a = ld(2 * i)
                    b = ld(2 * i + 1)
                    his.append(jnp.maximum(a, b))
                    los.append(jnp.maximum(a & lo_mask, b & lo_mask))
                while len(his) > 1:
                    his = [
                        jnp.maximum(his[2 * i], his[2 * i + 1])
                        for i in range(len(his) // 2)
                    ]
                    los = [
                        jnp.maximum(los[2 * i], los[2 * i + 1])
                        for i in range(len(los) // 2)
                    ]
                word = (his[0] & hi_mask) | los[0]
                obuf[q * NB + g, pl.ds(col, 16)] = plsc.bitcast(
                    word, jnp.int32
                )

    pltpu.async_copy(gather_src(0), gbufs[0], gsems[0])
    pltpu.async_copy(gather_src(1), gbufs[1], gsems[1])

    def outer(g2, _):
        for gg in range(2):
            g = g2 * 2 + gg
            obuf, osem = obufs[gg], osems[gg]

            @pl.when(g >= 2)
            def _drain_out():
                pltpu.make_async_copy(obuf, out_dst(g - 2), osem).wait()

            for q in range(4):
                j = g * 4 + q
                gb, gs = gbufs[q % 2], gsems[q % 2]
                pltpu.make_async_copy(gather_src(j), gb, gs).wait()
                compute(gb, obuf, q)

                @pl.when(j + 2 < NBATCH)
                def _start_next():
                    pltpu.async_copy(gather_src(j + 2), gb, gs)

            pltpu.async_copy(obuf, out_dst(g), osem)
        return _

    NGROUP = NBATCH // 4
    lax.fori_loop(0, NGROUP // 2, outer, None)
    for gg in range(2):
        pltpu.make_async_copy(
            obufs[gg], out_dst(NGROUP - 2 + gg), osems[gg]
        ).wait()


def _to_key(u16):
    sign = u16 >> jnp.uint16(15)
    mask = jnp.uint16(0x8000) + sign * jnp.uint16(0x7FFF)
    return u16 ^ mask


def _from_key(key):
    pos = key >> jnp.uint16(15)
    mask = jnp.uint16(0xFFFF) - pos * jnp.uint16(0x7FFF)
    return key ^ mask


def kernel(x, neighborhood):
    xt = x.reshape(D, N_IN).T.reshape(N_IN, NC, DH).transpose(1, 0, 2)
    xt = jnp.pad(xt, ((0, 0), (0, N_IN_PAD - N_IN), (0, 0)))
    xt = xt.astype(jnp.bfloat16)
    keys = _to_key(jax.lax.bitcast_convert_type(xt, jnp.uint16))
    xt = jax.lax.bitcast_convert_type(
        keys.reshape(NC, N_IN_PAD, DW, 2), jnp.int32
    )
    idx = jnp.zeros((OUT_PAD, K), jnp.int32)
    idx = idx.at[:N_OUT].set(neighborhood.astype(jnp.int32))
    out_w = _sc_gather_max(xt, idx.reshape(-1))
    out_k = jax.lax.bitcast_convert_type(out_w, jnp.uint16)
    out_t = jax.lax.bitcast_convert_type(_from_key(out_k), jnp.bfloat16)
    out_t = out_t.reshape(NC, OUT_PAD, DH).transpose(1, 0, 2)
    out_t = out_t.reshape(OUT_PAD, D)[:N_OUT]
    return out_t.astype(jnp.float32).T.reshape(B, F, N_OUT)


# speedup vs baseline: 1.0737x; 1.0737x over previous
import functools

import jax
import jax.numpy as jnp
from jax import lax
from jax.experimental import pallas as pl
from jax.experimental.pallas import tpu as pltpu
from jax.experimental.pallas import tpu_sc as plsc

B = 2
F = 128
N_IN = 10000
N_OUT = 5000
K = 32
D = B * F
NC = 2
NS = 16
DH = D // NC
DW = DH // 2
OUT_PAD = 5120
PER_S = OUT_PAD // NS
NB = 4
ROWS = NB * K
NBATCH = PER_S // NB
OGROUP = 4 * NB
STRIPE = 624
TAIL = N_IN - STRIPE * NS


_mesh = plsc.VectorSubcoreMesh(core_axis_name="c", subcore_axis_name="s")


@functools.partial(
    pl.kernel,
    out_type=jax.ShapeDtypeStruct((NC, OUT_PAD, DW), jnp.int32),
    mesh=_mesh,
    scratch_types=[
        pltpu.VMEM((PER_S * K,), jnp.int32),
        pltpu.VMEM((ROWS, DW), jnp.int32),
        pltpu.VMEM((ROWS, DW), jnp.int32),
        pltpu.VMEM((OGROUP, DW), jnp.int32),
        pltpu.VMEM((OGROUP, DW), jnp.int32),
        pltpu.VMEM_SHARED((N_IN, DW), jnp.int32),
        pltpu.SemaphoreType.DMA,
        pltpu.SemaphoreType.DMA,
        pltpu.SemaphoreType.DMA,
        pltpu.SemaphoreType.DMA,
    ],
)
def _sc_gather_max(xt_hbm, idx_hbm, out_hbm, idx_v, gbuf0, gbuf1, obuf0,
                   obuf1, table, gsem0, gsem1, osem0, osem1):
    c = lax.axis_index("c")
    s = lax.axis_index("s")

    pltpu.sync_copy(
        xt_hbm.at[c, pl.ds(s * STRIPE, STRIPE), :],
        table.at[pl.ds(s * STRIPE, STRIPE)],
    )

    @pl.when(s == NS - 1)
    def _stage_tail():
        pltpu.sync_copy(
            xt_hbm.at[c, pl.ds(NS * STRIPE, TAIL), :],
            table.at[pl.ds(NS * STRIPE, TAIL)],
        )
    pltpu.sync_copy(idx_hbm.at[pl.ds(s * (PER_S * K), PER_S * K)], idx_v)
    plsc.subcore_barrier()

    gbufs = (gbuf0, gbuf1)
    gsems = (gsem0, gsem1)
    obufs = (obuf0, obuf1)
    osems = (osem0, osem1)

    def gather_src(j):
        return table.at[idx_v.at[pl.ds(j * ROWS, ROWS)]]

    def out_dst(g):
        return out_hbm.at[c, pl.ds(s * PER_S + g * OGROUP, OGROUP), :]

    def compute(buf, obuf, q):
        lo_mask = jnp.uint32(0xFFFF)
        hi_mask = jnp.uint32(0xFFFF0000)

        @plsc.parallel_loop(0, DW // 16)
        def col_body(d):
            col = d * 16
            for g in range(NB):
                r0 = g * K

                def ld(r):
                    return plsc.bitcast(
                        buf[r0 + r, pl.ds(col, 16)], jnp.uint32
                    )

                his = []
                los = []
                for i in range(K // 2):
                    a = ld(2 * i)
                    b = ld(2 * i + 1)
                    his.append(jnp.maximum(a, b))
                    los.append(jnp.maximum(a & lo_mask, b & lo_mask))
                while len(his) > 1:
                    his = [
                        jnp.maximum(his[2 * i], his[2 * i + 1])
                        for i in range(len(his) // 2)
                    ]
                    los = [
                        jnp.maximum(los[2 * i], los[2 * i + 1])
                        for i in range(len(los) // 2)
                    ]
                word = (his[0] & hi_mask) | los[0]
                obuf[q * NB + g, pl.ds(col, 16)] = plsc.bitcast(
                    word, jnp.int32
                )

    pltpu.async_copy(gather_src(0), gbufs[0], gsems[0])
    pltpu.async_copy(gather_src(1), gbufs[1], gsems[1])

    def outer(g2, _):
        for gg in range(2):
            g = g2 * 2 + gg
            obuf, osem = obufs[gg], osems[gg]

            @pl.when(g >= 2)
            def _drain_out():
                pltpu.make_async_copy(obuf, out_dst(g - 2), osem).wait()

            for q in range(4):
                j = g * 4 + q
                gb, gs = gbufs[q % 2], gsems[q % 2]
                pltpu.make_async_copy(gather_src(j), gb, gs).wait()
                compute(gb, obuf, q)

                @pl.when(j + 2 < NBATCH)
                def _start_next():
                    pltpu.async_copy(gather_src(j + 2), gb, gs)

            pltpu.async_copy(obuf, out_dst(g), osem)
        return _

    NGROUP = NBATCH // 4
    lax.fori_loop(0, NGROUP // 2, outer, None)
    for gg in range(2):
        pltpu.make_async_copy(
            obufs[gg], out_dst(NGROUP - 2 + gg), osems[gg]
        ).wait()


def _to_key_words(w):
    shi = (w >> jnp.uint32(31)) & jnp.uint32(1)
    slo = (w >> jnp.uint32(15)) & jnp.uint32(1)
    mask = (
        jnp.uint32(0x80008000)
        + shi * jnp.uint32(0x7FFF0000)
        + slo * jnp.uint32(0x7FFF)
    )
    return w ^ mask


def _from_key_words(w):
    phi = (w >> jnp.uint32(31)) & jnp.uint32(1)
    plo = (w >> jnp.uint32(15)) & jnp.uint32(1)
    mask = (
        jnp.uint32(0xFFFFFFFF)
        - phi * jnp.uint32(0x7FFF0000)
        - plo * jnp.uint32(0x7FFF)
    )
    return w ^ mask


def kernel(x, neighborhood):
    xt = x.transpose(0, 2, 1).astype(jnp.bfloat16)
    w = jax.lax.bitcast_convert_type(
        xt.reshape(NC, N_IN, DW, 2), jnp.uint32
    )
    w = jax.lax.bitcast_convert_type(_to_key_words(w), jnp.int32)
    idx = jnp.zeros((OUT_PAD, K), jnp.int32)
    idx = idx.at[:N_OUT].set(neighborhood.astype(jnp.int32))
    out_w = _sc_gather_max(w, idx.reshape(-1))
    out_w = _from_key_words(jax.lax.bitcast_convert_type(out_w, jnp.uint32))
    out_t = jax.lax.bitcast_convert_type(out_w, jnp.bfloat16)
    out_t = out_t.reshape(NC, OUT_PAD, DH)[:, :N_OUT, :]
    return out_t.astype(jnp.float32).transpose(0, 2, 1)


# speedup vs baseline: 1.3553x; 1.2623x over previous
import functools

import jax
import jax.numpy as jnp
from jax import lax
from jax.experimental import pallas as pl
from jax.experimental.pallas import tpu as pltpu
from jax.experimental.pallas import tpu_sc as plsc

B = 2
F = 128
N_IN = 10000
N_OUT = 5000
K = 32
D = B * F
NC = 2
NS = 16
DH = D // NC
DW = DH // 2
OUT_PAD = 5120
PER_S = OUT_PAD // NS
NB = 4
ROWS = NB * K
NBATCH = PER_S // NB
OGROUP = 4 * NB
STRIPE = 624
TAIL = N_IN - STRIPE * NS


_mesh = plsc.VectorSubcoreMesh(core_axis_name="c", subcore_axis_name="s")


@functools.partial(
    pl.kernel,
    out_type=jax.ShapeDtypeStruct((NC, OUT_PAD, DW), jnp.int32),
    mesh=_mesh,
    scratch_types=[
        pltpu.VMEM((PER_S * K,), jnp.int32),
        pltpu.VMEM((ROWS, DW), jnp.int32),
        pltpu.VMEM((ROWS, DW), jnp.int32),
        pltpu.VMEM((OGROUP, DW), jnp.int32),
        pltpu.VMEM((OGROUP, DW), jnp.int32),
        pltpu.VMEM_SHARED((N_IN, DW), jnp.int32),
        pltpu.SemaphoreType.DMA,
        pltpu.SemaphoreType.DMA,
        pltpu.SemaphoreType.DMA,
        pltpu.SemaphoreType.DMA,
    ],
)
def _sc_gather_max(xt_hbm, idx_hbm, out_hbm, idx_v, gbuf0, gbuf1, obuf0,
                   obuf1, table, gsem0, gsem1, osem0, osem1):
    c = lax.axis_index("c")
    s = lax.axis_index("s")

    pltpu.sync_copy(
        xt_hbm.at[c, pl.ds(s * STRIPE, STRIPE), :],
        table.at[pl.ds(s * STRIPE, STRIPE)],
    )

    @pl.when(s == NS - 1)
    def _stage_tail():
        pltpu.sync_copy(
            xt_hbm.at[c, pl.ds(NS * STRIPE, TAIL), :],
            table.at[pl.ds(NS * STRIPE, TAIL)],
        )
    pltpu.sync_copy(idx_hbm.at[pl.ds(s * (PER_S * K), PER_S * K)], idx_v)
    plsc.subcore_barrier()

    gbufs = (gbuf0, gbuf1)
    gsems = (gsem0, gsem1)
    obufs = (obuf0, obuf1)
    osems = (osem0, osem1)

    def gather_src(j):
        return table.at[idx_v.at[pl.ds(j * ROWS, ROWS)]]

    def out_dst(g):
        return out_hbm.at[c, pl.ds(s * PER_S + g * OGROUP, OGROUP), :]

    def compute(buf, obuf, q):
        lo_mask = jnp.uint32(0xFFFF)
        hi_mask = jnp.uint32(0xFFFF0000)

        @plsc.parallel_loop(0, DW // 16)
        def col_body(d):
            col = d * 16
            for g in range(NB):
                r0 = g * K

                def ld(r):
                    return plsc.bitcast(
                        buf[r0 + r, pl.ds(col, 16)], jnp.uint32
                    )

                his = []
                los = []
                for i in range(K // 2):
                    a = ld(2 * i)
                    b = ld(2 * i + 1)
                    his.append(jnp.maximum(a, b))
                    los.append(jnp.maximum(a & lo_mask, b & lo_mask))
                while len(his) > 1:
                    his = [
                        jnp.maximum(his[2 * i], his[2 * i + 1])
                        for i in range(len(his) // 2)
                    ]
                    los = [
                        jnp.maximum(los[2 * i], los[2 * i + 1])
                        for i in range(len(los) // 2)
                    ]
                word = (his[0] & hi_mask) | los[0]
                obuf[q * NB + g, pl.ds(col, 16)] = plsc.bitcast(
                    word, jnp.int32
                )

    pltpu.async_copy(gather_src(0), gbufs[0], gsems[0])
    pltpu.async_copy(gather_src(1), gbufs[1], gsems[1])

    def outer(g2, _):
        for gg in range(2):
            g = g2 * 2 + gg
            obuf, osem = obufs[gg], osems[gg]

            @pl.when(g >= 2)
            def _drain_out():
                pltpu.make_async_copy(obuf, out_dst(g - 2), osem).wait()

            for q in range(4):
                j = g * 4 + q
                gb, gs = gbufs[q % 2], gsems[q % 2]
                pltpu.make_async_copy(gather_src(j), gb, gs).wait()
                compute(gb, obuf, q)

                @pl.when(j + 2 < NBATCH)
                def _start_next():
                    pltpu.async_copy(gather_src(j + 2), gb, gs)

            pltpu.async_copy(obuf, out_dst(g), osem)
        return _

    NGROUP = NBATCH // 4
    lax.fori_loop(0, NGROUP // 2, outer, None)
    for gg in range(2):
        pltpu.make_async_copy(
            obufs[gg], out_dst(NGROUP - 2 + gg), osems[gg]
        ).wait()


def _to_key_words(w):
    shi = (w >> jnp.uint32(31)) & jnp.uint32(1)
    slo = (w >> jnp.uint32(15)) & jnp.uint32(1)
    mask = (
        jnp.uint32(0x80008000)
        + shi * jnp.uint32(0x7FFF0000)
        + slo * jnp.uint32(0x7FFF)
    )
    return w ^ mask


def _from_key_words(w):
    phi = (w >> jnp.uint32(31)) & jnp.uint32(1)
    plo = (w >> jnp.uint32(15)) & jnp.uint32(1)
    mask = (
        jnp.uint32(0xFFFFFFFF)
        - phi * jnp.uint32(0x7FFF0000)
        - plo * jnp.uint32(0x7FFF)
    )
    return w ^ mask


def kernel(x, neighborhood):
    u = jax.lax.bitcast_convert_type(x, jnp.uint32)
    r = (u + jnp.uint32(0x7FFF) + ((u >> jnp.uint32(16)) & jnp.uint32(1))
         ) >> jnp.uint32(16)
    w = r[0] | (r[1] << jnp.uint32(16))
    w = _to_key_words(w)
    table = jax.lax.bitcast_convert_type(
        w.reshape(NC, DW, N_IN).transpose(0, 2, 1), jnp.int32
    )
    idx = jnp.zeros((OUT_PAD, K), jnp.int32)
    idx = idx.at[:N_OUT].set(neighborhood.astype(jnp.int32))
    out_w = _sc_gather_max(table, idx.reshape(-1))
    out_w = _from_key_words(jax.lax.bitcast_convert_type(out_w, jnp.uint32))
    out_w = out_w.transpose(0, 2, 1).reshape(F, OUT_PAD)[:, :N_OUT]
    lo = jax.lax.bitcast_convert_type(out_w << jnp.uint32(16), jnp.float32)
    hi = jax.lax.bitcast_convert_type(
        out_w & jnp.uint32(0xFFFF0000), jnp.float32
    )
    return jnp.stack([lo, hi])
